# baseline (speedup 1.0000x reference)
import functools

import jax
import jax.numpy as jnp
from jax.experimental import pallas as pl
from jax.experimental.pallas import tpu as pltpu


def _gru_kernel(x_ref, emb_ref, W_ih_ref, W_hh_ref, b_comb_ref, b_hhn_ref,
                W_fc_ref, b_fc_ref, out_ref, hT_ref, GT_ref, Whh_ref, Wfc_ref,
                *, H, U):
    t = pl.program_id(0)
    B = hT_ref.shape[1]
    V = Wfc_ref.shape[0]

    @pl.when(t == 0)
    def _init():
        Whh_ref[...] = W_hh_ref[...].astype(jnp.bfloat16)
        Wfc_ref[...] = W_fc_ref[...].astype(jnp.bfloat16)
        GT_ref[...] = (
            jax.lax.dot_general(W_ih_ref[...], emb_ref[...],
                                (((1,), (1,)), ((), ())),
                                preferred_element_type=jnp.float32)
            + b_comb_ref[...]
        ).astype(jnp.bfloat16)
        hT_ref[...] = jnp.zeros_like(hT_ref)

    hT = hT_ref[...]
    logits = []
    for u in range(U):
        idx = x_ref[t * U + u]
        onehotT = (jax.lax.broadcasted_iota(jnp.int32, (V, B), 0)
                   == idx[None, :]).astype(jnp.bfloat16)
        giT = jnp.dot(GT_ref[...], onehotT, preferred_element_type=jnp.float32)

        ghT = jnp.dot(Whh_ref[...], hT.astype(jnp.bfloat16),
                      preferred_element_type=jnp.float32)

        r = jax.nn.sigmoid(giT[:H] + ghT[:H])
        z = jax.nn.sigmoid(giT[H:2 * H] + ghT[H:2 * H])
        n = jnp.tanh(giT[2 * H:] + r * (ghT[2 * H:] + b_hhn_ref[...]))
        hT = (1.0 - z) * n + z * hT

        logits.append(
            (jnp.dot(Wfc_ref[...], hT.astype(jnp.bfloat16),
                     preferred_element_type=jnp.float32)
             + b_fc_ref[...]).T
        )
    hT_ref[...] = hT
    out_ref[...] = jnp.stack(logits, axis=1)


def kernel(x_in, emb, W_ih, W_hh, b_ih, b_hh, W_fc, b_fc):
    B, S = x_in.shape
    V, E = emb.shape
    H = W_hh.shape[1]

    x = x_in.astype(jnp.int32).T
    b_comb = (b_ih + jnp.concatenate(
        [b_hh[:2 * H], jnp.zeros_like(b_hh[2 * H:])])).reshape(-1, 1)
    b_hhn = b_hh[2 * H:].reshape(-1, 1)
    b_fc2 = b_fc.reshape(-1, 1)

    U = 8
    full = lambda shape: pl.BlockSpec(shape, lambda t: (0,) * len(shape))
    out = pl.pallas_call(
        functools.partial(_gru_kernel, H=H, U=U),
        grid=(S // U,),
        in_specs=[
            full((S, B)),
            full((V, E)),
            full((3 * H, E)),
            full((3 * H, H)),
            full((3 * H, 1)),
            full((H, 1)),
            full((V, H)),
            full((V, 1)),
        ],
        out_specs=pl.BlockSpec((B, U, V), lambda t: (0, t, 0)),
        out_shape=jax.ShapeDtypeStruct((B, S, V), jnp.float32),
        scratch_shapes=[
            pltpu.VMEM((H, B), jnp.float32),
            pltpu.VMEM((3 * H, V), jnp.bfloat16),
            pltpu.VMEM((3 * H, H), jnp.bfloat16),
            pltpu.VMEM((V, H), jnp.bfloat16),
        ],
    )(x, emb, W_ih, W_hh, b_comb, b_hhn, W_fc, b_fc2)
    return out


# speedup vs baseline: 1.6771x; 1.6771x over previous
import functools

import jax
import jax.numpy as jnp
from jax.experimental import pallas as pl
from jax.experimental.pallas import tpu as pltpu


def _gru_kernel(x_ref, emb_ref, W_ih_ref, W_hh_ref, b_comb_ref, b_hhn_ref,
                W_fc_ref, b_fc_ref, out_ref, h_ref, G_ref, Whh_ref, Wfc_ref,
                gi_ref, hU_ref, *, H, U):
    t = pl.program_id(0)
    B = h_ref.shape[0]
    V = G_ref.shape[0]

    @pl.when(t == 0)
    def _init():
        Whh_ref[...] = W_hh_ref[...].astype(jnp.bfloat16).T
        Wfc_ref[...] = W_fc_ref[...].astype(jnp.bfloat16).T
        G_ref[...] = (
            jax.lax.dot_general(emb_ref[...], W_ih_ref[...],
                                (((1,), (1,)), ((), ())),
                                preferred_element_type=jnp.float32)
            + b_comb_ref[...]
        ).astype(jnp.bfloat16)
        h_ref[...] = jnp.zeros_like(h_ref)

    iota_v = jax.lax.broadcasted_iota(jnp.int32, (B, V), 1)
    onehotU = jnp.concatenate(
        [(x_ref[t * U + u][:, None] == iota_v).astype(jnp.bfloat16)
         for u in range(U)], axis=0)
    gi_ref[...] = jnp.dot(onehotU, G_ref[...],
                          preferred_element_type=jnp.float32
                          ).astype(jnp.bfloat16)

    h = h_ref[...]
    for u in range(U):
        gi = gi_ref[u * B:(u + 1) * B, :].astype(jnp.float32)

        gh = jnp.dot(h.astype(jnp.bfloat16), Whh_ref[...],
                     preferred_element_type=jnp.float32)

        r = jax.nn.sigmoid(gi[:, :H] + gh[:, :H])
        z = jax.nn.sigmoid(gi[:, H:2 * H] + gh[:, H:2 * H])
        n = jnp.tanh(gi[:, 2 * H:] + r * (gh[:, 2 * H:] + b_hhn_ref[...]))
        h = (1.0 - z) * n + z * h

        hU_ref[u * B:(u + 1) * B, :] = h.astype(jnp.bfloat16)
    h_ref[...] = h

    logitsU = (jnp.dot(hU_ref[...], Wfc_ref[...],
                       preferred_element_type=jnp.float32)
               + b_fc_ref[...])
    out_ref[...] = jnp.swapaxes(logitsU.reshape(U, B, V), 0, 1)


def kernel(x_in, emb, W_ih, W_hh, b_ih, b_hh, W_fc, b_fc):
    B, S = x_in.shape
    V, E = emb.shape
    H = W_hh.shape[1]

    x = x_in.astype(jnp.int32).T
    b_comb = (b_ih + jnp.concatenate(
        [b_hh[:2 * H], jnp.zeros_like(b_hh[2 * H:])])).reshape(1, -1)
    b_hhn = b_hh[2 * H:].reshape(1, -1)
    b_fc2 = b_fc.reshape(1, -1)

    U = 8
    full = lambda shape: pl.BlockSpec(shape, lambda t: (0,) * len(shape))
    out = pl.pallas_call(
        functools.partial(_gru_kernel, H=H, U=U),
        grid=(S // U,),
        in_specs=[
            full((S, B)),
            full((V, E)),
            full((3 * H, E)),
            full((3 * H, H)),
            full((1, 3 * H)),
            full((1, H)),
            full((V, H)),
            full((1, V)),
        ],
        out_specs=pl.BlockSpec((B, U, V), lambda t: (0, t, 0)),
        out_shape=jax.ShapeDtypeStruct((B, S, V), jnp.float32),
        scratch_shapes=[
            pltpu.VMEM((B, H), jnp.float32),
            pltpu.VMEM((V, 3 * H), jnp.bfloat16),
            pltpu.VMEM((H, 3 * H), jnp.bfloat16),
            pltpu.VMEM((H, V), jnp.bfloat16),
            pltpu.VMEM((U * B, 3 * H), jnp.bfloat16),
            pltpu.VMEM((U * B, H), jnp.bfloat16),
        ],
    )(x, emb, W_ih, W_hh, b_comb, b_hhn, W_fc, b_fc2)
    return out


# speedup vs baseline: 1.8025x; 1.0748x over previous
import functools

import jax
import jax.numpy as jnp
from jax.experimental import pallas as pl
from jax.experimental.pallas import tpu as pltpu


def _gru_kernel(x_ref, emb_ref, W_ih_ref, W_hh_ref, b_comb_ref, b_hhn_ref,
                W_fc_ref, b_fc_ref, out_ref, h_ref, G_ref, Whh_ref, Wfc_ref,
                *, H, U):
    t = pl.program_id(0)
    B = h_ref.shape[0]
    V = G_ref.shape[0]

    @pl.when(t == 0)
    def _init():
        Whh_ref[...] = W_hh_ref[...].astype(jnp.bfloat16).T
        Wfc_ref[...] = W_fc_ref[...].astype(jnp.bfloat16).T
        G_ref[...] = (
            jax.lax.dot_general(emb_ref[...], W_ih_ref[...],
                                (((1,), (1,)), ((), ())),
                                preferred_element_type=jnp.float32)
            + b_comb_ref[...]
        ).astype(jnp.bfloat16)
        h_ref[...] = jnp.zeros_like(h_ref)

    h = h_ref[...]
    hs = []
    for u in range(U):
        idx = x_ref[t * U + u]
        onehot = (idx[:, None]
                  == jax.lax.broadcasted_iota(jnp.int32, (B, V), 1)
                  ).astype(jnp.bfloat16)
        gi = jnp.dot(onehot, G_ref[...], preferred_element_type=jnp.float32)

        gh = jnp.dot(h.astype(jnp.bfloat16), Whh_ref[...],
                     preferred_element_type=jnp.float32)

        r = jax.nn.sigmoid(gi[:, :H] + gh[:, :H])
        z = jax.nn.sigmoid(gi[:, H:2 * H] + gh[:, H:2 * H])
        n = jnp.tanh(gi[:, 2 * H:] + r * (gh[:, 2 * H:] + b_hhn_ref[...]))
        h = (1.0 - z) * n + z * h

        hs.append(h.astype(jnp.bfloat16))
    h_ref[...] = h

    logitsU = (jnp.dot(jnp.concatenate(hs, axis=0), Wfc_ref[...],
                       preferred_element_type=jnp.float32)
               + b_fc_ref[...])
    out_ref[...] = jnp.swapaxes(logitsU.reshape(U, B, V), 0, 1)


def kernel(x_in, emb, W_ih, W_hh, b_ih, b_hh, W_fc, b_fc):
    B, S = x_in.shape
    V, E = emb.shape
    H = W_hh.shape[1]

    x = x_in.astype(jnp.int32).T
    b_comb = (b_ih + jnp.concatenate(
        [b_hh[:2 * H], jnp.zeros_like(b_hh[2 * H:])])).reshape(1, -1)
    b_hhn = b_hh[2 * H:].reshape(1, -1)
    b_fc2 = b_fc.reshape(1, -1)

    U = 8
    full = lambda shape: pl.BlockSpec(shape, lambda t: (0,) * len(shape))
    out = pl.pallas_call(
        functools.partial(_gru_kernel, H=H, U=U),
        grid=(S // U,),
        in_specs=[
            full((S, B)),
            full((V, E)),
            full((3 * H, E)),
            full((3 * H, H)),
            full((1, 3 * H)),
            full((1, H)),
            full((V, H)),
            full((1, V)),
        ],
        out_specs=pl.BlockSpec((B, U, V), lambda t: (0, t, 0)),
        out_shape=jax.ShapeDtypeStruct((B, S, V), jnp.float32),
        scratch_shapes=[
            pltpu.VMEM((B, H), jnp.float32),
            pltpu.VMEM((V, 3 * H), jnp.bfloat16),
            pltpu.VMEM((H, 3 * H), jnp.bfloat16),
            pltpu.VMEM((H, V), jnp.bfloat16),
        ],
    )(x, emb, W_ih, W_hh, b_comb, b_hhn, W_fc, b_fc2)
    return out
